# Initial kernel scaffold; baseline (speedup 1.0000x reference)
import jax
import jax.numpy as jnp
from jax.experimental import pallas as pl


def kernel(uid, utag1, utag2, utag3, utag4, label_goods_ids, label_shop_ids, label_cate_ids, longterm_goods_ids, longterm_shop_ids, longterm_cate_ids, shortterm_goods_ids, shortterm_shop_ids, shortterm_cate_ids, embed_table, H, st_wq, st_bq, st_wk, st_bk, st_wv, st_bv, st_wo, st_bo, lt_wq, lt_bq, lt_wk, lt_bk, lt_wv, lt_bv, lt_wo, lt_bo, W1, b1, g1, be1, W2, b2, g2, be2, W3, b3):
    raise NotImplementedError("write your pallas kernel here")


# speedup vs baseline: 5.5455x; 5.5455x over previous
import functools
import math

import jax
import jax.numpy as jnp
from jax import lax
from jax.experimental import pallas as pl
from jax.experimental.pallas import tpu as pltpu
from jax.experimental.pallas import tpu_sc as plsc

B = 1024
E = 64
L_LONG = 200
L_SHORT = 50
LSH_DIM = 16
TOPK = 16
NH = 8
KD = 32
DQK = NH * KD
D_ITEM = 3 * E
D_USER = 5 * E

_NC = 2
_NS = 16
_NW = _NC * _NS

_CHUNK = 128
_NFIRE = 8
_SUPER = _CHUNK * _NFIRE

_BB = 32


def _sc_gather(table, idx2):
    n = idx2.shape[0] * _CHUNK
    d = table.shape[1]
    per_w = n // _NW
    n_super = per_w // _SUPER
    mesh = plsc.VectorSubcoreMesh(core_axis_name="c", subcore_axis_name="s")

    @functools.partial(
        pl.kernel,
        mesh=mesh,
        compiler_params=pltpu.CompilerParams(use_tc_tiling_on_sc=False),
        out_type=jax.ShapeDtypeStruct((n, d), jnp.float32),
        scratch_types=[
            pltpu.VMEM((_NFIRE, _CHUNK), jnp.int32),
            pltpu.VMEM((_SUPER, d), jnp.float32),
            pltpu.SemaphoreType.DMA,
        ],
    )
    def gath(table_hbm, idx_hbm, out_hbm, idx_v, rows_v, gsem):
        wid = lax.axis_index("s") * _NC + lax.axis_index("c")
        base = wid * per_w

        def body(i, carry):
            off = pl.multiple_of(base + i * _SUPER, _SUPER)
            pltpu.sync_copy(
                idx_hbm.at[pl.ds(pl.multiple_of(off // _CHUNK, _NFIRE),
                                 _NFIRE)], idx_v)
            handles = []
            for j in range(_NFIRE):
                handles.append(
                    pltpu.async_copy(
                        table_hbm.at[idx_v.at[j]],
                        rows_v.at[pl.ds(j * _CHUNK, _CHUNK)],
                        gsem,
                    )
                )
            for h in handles:
                h.wait()
            pltpu.sync_copy(rows_v, out_hbm.at[pl.ds(off, _SUPER)])
            return carry

        lax.fori_loop(0, n_super, body, 0)

    return gath(table, idx2)


def _tc_body(xi_ref, xu_ref, xs_ref, xl_ref, stg_ref, ltg_ref, h_ref,
             sqw_ref, sqb_ref, skw_ref, skb_ref, svw_ref, svb_ref,
             sow_ref, sob_ref,
             lqw_ref, lqb_ref, lkw_ref, lkb_ref, lvw_ref, lvb_ref,
             low_ref, lob_ref,
             w1_ref, b1_ref, g1_ref, be1_ref,
             w2_ref, b2_ref, g2_ref, be2_ref,
             w3_ref, b3_ref, out_ref):
    f32 = jnp.float32
    bB = _BB
    Xi = xi_ref[...]
    Xu = xu_ref[...]
    Xs = xs_ref[...]
    Xl = xl_ref[...]

    ki = lax.broadcasted_iota(jnp.int32, (DQK, NH), 0)
    hi = lax.broadcasted_iota(jnp.int32, (DQK, NH), 1)
    S = (ki // KD == hi).astype(f32)
    ki2 = lax.broadcasted_iota(jnp.int32, (NH, DQK), 1)
    hi2 = lax.broadcasted_iota(jnp.int32, (NH, DQK), 0)
    ST = (ki2 // KD == hi2).astype(f32)

    def mm(a, b):
        return jnp.dot(a, b, preferred_element_type=f32)

    def mha(X, L, pen3, qw, qb, kw, kb, vw, vb, ow, ob):
        qh = mm(Xi, qw) + qb
        kh = mm(X, kw) + kb
        vh = mm(X, vw) + vb
        qr = jnp.broadcast_to(qh.reshape(bB, 1, DQK), (bB, L, DQK))
        qr = qr.reshape(bB * L, DQK)
        sc = mm(kh * qr, S) * (1.0 / math.sqrt(float(KD)))
        sc3 = sc.reshape(bB, L, NH) + pen3
        m = jnp.max(sc3, axis=1, keepdims=True)
        e = jnp.exp(sc3 - m)
        a = e / jnp.sum(e, axis=1, keepdims=True)
        ar = mm(a.reshape(bB * L, NH), ST)
        o = jnp.sum((ar * vh).reshape(bB, L, DQK), axis=1)
        return mm(o, ow) + ob

    svalid = (stg_ref[...] != 0)
    stlen = jnp.sum(svalid.astype(jnp.int32), axis=1, keepdims=True)
    t3 = lax.broadcasted_iota(jnp.int32, (bB, L_SHORT, 1), 1)
    pen_s = jnp.where(t3 < stlen, 0.0, -1e9).astype(f32)
    st_int = mha(Xs, L_SHORT, pen_s,
                 sqw_ref[...], sqb_ref[...], skw_ref[...], skb_ref[...],
                 svw_ref[...], svb_ref[...], sow_ref[...], sob_ref[...])

    H = h_ref[...]
    ic = jnp.sign(mm(Xi, H))
    lc = jnp.sign(mm(Xl, H))
    icr = jnp.broadcast_to(ic.reshape(bB, 1, LSH_DIM), (bB, L_LONG, LSH_DIM))
    eq = (lc.reshape(bB, L_LONG, LSH_DIM) == icr).astype(f32)
    s3 = jnp.sum(eq, axis=2, keepdims=True)
    lvalid = (ltg_ref[...] != 0)
    s3 = jnp.where(lvalid, s3, -1.0)
    l3 = lax.broadcasted_iota(jnp.int32, (bB, L_LONG, 1), 1)
    sel = jnp.zeros((bB, L_LONG, 1), jnp.bool_)
    work = s3
    for _ in range(TOPK):
        mx = jnp.max(work, axis=1, keepdims=True)
        cand = jnp.where(work == mx, l3, L_LONG)
        imin = jnp.min(cand, axis=1, keepdims=True)
        oh = (l3 == imin)
        sel = jnp.logical_or(sel, oh)
        work = jnp.where(oh, -10.0, work)
    pen_l = jnp.where(sel, jnp.where(lvalid, 0.0, -1e9), -2e9).astype(f32)
    lt_int = mha(Xl, L_LONG, pen_l,
                 lqw_ref[...], lqb_ref[...], lkw_ref[...], lkb_ref[...],
                 lvw_ref[...], lvb_ref[...], low_ref[...], lob_ref[...])

    def layernorm(x, g, b):
        mu = jnp.mean(x, axis=1, keepdims=True)
        var = jnp.mean((x - mu) ** 2, axis=1, keepdims=True)
        return (x - mu) / jnp.sqrt(var + 1e-3) * g + b

    xcomb = jnp.concatenate([Xi, Xu, st_int, lt_int], axis=1)
    h1 = jnp.maximum(layernorm(mm(xcomb, w1_ref[...]) + b1_ref[...],
                               g1_ref[...], be1_ref[...]), 0.0)
    h2 = jnp.maximum(layernorm(mm(h1, w2_ref[...]) + b2_ref[...],
                               g2_ref[...], be2_ref[...]), 0.0)
    out_ref[...] = jax.nn.sigmoid(mm(h2, w3_ref[...]) + b3_ref[...])


def _tc_forward(Xi, Xu, Xs, Xl, stg3, ltg3, H,
                sqw, sqb, skw, skb, svw, svb, sow, sob,
                lqw, lqb, lkw, lkb, lvw, lvb, low, lob,
                W1, b1, g1, be1, W2, b2, g2, be2, W3, b3):
    bB = _BB
    grid = (B // bB,)

    def row(i):
        return (i, 0)

    def row3(i):
        return (i, 0, 0)

    def full(i):
        return (0, 0)

    in_specs = [
        pl.BlockSpec((bB, D_ITEM), row),
        pl.BlockSpec((bB, D_USER), row),
        pl.BlockSpec((bB * L_SHORT, D_ITEM), row),
        pl.BlockSpec((bB * L_LONG, D_ITEM), row),
        pl.BlockSpec((bB, L_SHORT, 1), row3),
        pl.BlockSpec((bB, L_LONG, 1), row3),
        pl.BlockSpec((D_ITEM, LSH_DIM), full),
    ]
    for _ in range(2):
        in_specs += [
            pl.BlockSpec((D_ITEM, DQK), full), pl.BlockSpec((1, DQK), full),
            pl.BlockSpec((D_ITEM, DQK), full), pl.BlockSpec((1, DQK), full),
            pl.BlockSpec((D_ITEM, DQK), full), pl.BlockSpec((1, DQK), full),
            pl.BlockSpec((DQK, D_ITEM), full), pl.BlockSpec((1, D_ITEM), full),
        ]
    in_specs += [
        pl.BlockSpec((896, 200), full), pl.BlockSpec((1, 200), full),
        pl.BlockSpec((1, 200), full), pl.BlockSpec((1, 200), full),
        pl.BlockSpec((200, 80), full), pl.BlockSpec((1, 80), full),
        pl.BlockSpec((1, 80), full), pl.BlockSpec((1, 80), full),
        pl.BlockSpec((80, 1), full), pl.BlockSpec((1, 1), full),
    ]
    return pl.pallas_call(
        _tc_body,
        grid=grid,
        in_specs=in_specs,
        out_specs=pl.BlockSpec((bB, 1), row),
        out_shape=jax.ShapeDtypeStruct((B, 1), jnp.float32),
        compiler_params=pltpu.CompilerParams(
            dimension_semantics=("arbitrary",)),
    )(Xi, Xu, Xs, Xl, stg3, ltg3, H,
      sqw, sqb, skw, skb, svw, svb, sow, sob,
      lqw, lqb, lkw, lkb, lvw, lvb, low, lob,
      W1, b1, g1, be1, W2, b2, g2, be2, W3, b3)


def kernel(uid, utag1, utag2, utag3, utag4,
           label_goods_ids, label_shop_ids, label_cate_ids,
           longterm_goods_ids, longterm_shop_ids, longterm_cate_ids,
           shortterm_goods_ids, shortterm_shop_ids, shortterm_cate_ids,
           embed_table, H,
           st_wq, st_bq, st_wk, st_bk, st_wv, st_bv, st_wo, st_bo,
           lt_wq, lt_bq, lt_wk, lt_bk, lt_wv, lt_bv, lt_wo, lt_bo,
           W1, b1, g1, be1, W2, b2, g2, be2, W3, b3):
    i32 = jnp.int32
    lg = label_goods_ids.astype(i32)
    ls = label_shop_ids.astype(i32)
    lc = label_cate_ids.astype(i32)
    stg = shortterm_goods_ids.astype(i32)
    sts = shortterm_shop_ids.astype(i32)
    stc = shortterm_cate_ids.astype(i32)
    ltg = longterm_goods_ids.astype(i32)
    lts = longterm_shop_ids.astype(i32)
    ltc = longterm_cate_ids.astype(i32)

    idx_item = jnp.stack([lg, ls, lc], axis=1).reshape(-1)
    idx_user = jnp.stack([uid.astype(i32), utag1.astype(i32),
                          utag2.astype(i32), utag3.astype(i32),
                          utag4.astype(i32)], axis=1).reshape(-1)
    idx_short = jnp.stack([stg, sts, stc], axis=2).reshape(-1)
    idx_long = jnp.stack([ltg, lts, ltc], axis=2).reshape(-1)
    idx_all = jnp.concatenate([idx_item, idx_user, idx_short, idx_long])
    n0 = idx_all.shape[0]
    unit = _NW * _SUPER
    npad = ((n0 + unit - 1) // unit) * unit
    idx_all = jnp.concatenate(
        [idx_all, jnp.zeros((npad - n0,), i32)]).reshape(-1, _CHUNK)

    G = _sc_gather(embed_table, idx_all)

    o = 0
    Xi = G[o:o + 3 * B].reshape(B, D_ITEM); o += 3 * B
    Xu = G[o:o + 5 * B].reshape(B, D_USER); o += 5 * B
    Xs = G[o:o + B * 3 * L_SHORT].reshape(B * L_SHORT, D_ITEM)
    o += B * 3 * L_SHORT
    Xl = G[o:o + B * 3 * L_LONG].reshape(B * L_LONG, D_ITEM)

    r2 = lambda v, d: v.reshape(1, d)
    out = _tc_forward(
        Xi, Xu, Xs, Xl,
        stg.reshape(B, L_SHORT, 1), ltg.reshape(B, L_LONG, 1), H,
        st_wq.reshape(D_ITEM, DQK), r2(st_bq, DQK),
        st_wk.reshape(D_ITEM, DQK), r2(st_bk, DQK),
        st_wv.reshape(D_ITEM, DQK), r2(st_bv, DQK),
        st_wo.reshape(DQK, D_ITEM), r2(st_bo, D_ITEM),
        lt_wq.reshape(D_ITEM, DQK), r2(lt_bq, DQK),
        lt_wk.reshape(D_ITEM, DQK), r2(lt_bk, DQK),
        lt_wv.reshape(D_ITEM, DQK), r2(lt_bv, DQK),
        lt_wo.reshape(DQK, D_ITEM), r2(lt_bo, D_ITEM),
        W1, r2(b1, 200), r2(g1, 200), r2(be1, 200),
        W2, r2(b2, 80), r2(g2, 80), r2(be2, 80),
        W3, r2(b3, 1))
    return out


# speedup vs baseline: 9.2326x; 1.6649x over previous
import functools
import math

import jax
import jax.numpy as jnp
from jax import lax
from jax.experimental import pallas as pl
from jax.experimental.pallas import tpu as pltpu
from jax.experimental.pallas import tpu_sc as plsc

B = 1024
E = 64
L_LONG = 200
L_SHORT = 50
LSH_DIM = 16
TOPK = 16
NH = 8
KD = 32
DQK = NH * KD
D_ITEM = 3 * E
D_USER = 5 * E

_NC = 2
_NS = 16
_NW = _NC * _NS

_CHUNK = 128
_NFIRE = 8
_SUPER = _CHUNK * _NFIRE

_BB = 32


_KI = B // _NW
_KS = B * L_SHORT // _NW
_KL = B * L_LONG // _NW


def _sc_gather_all(table, ig, ish, ica, u0, u1, u2, u3, u4,
                   sg, ss, sc, lg, ls, lc):
    mesh = plsc.VectorSubcoreMesh(core_axis_name="c", subcore_axis_name="s")

    @functools.partial(
        pl.kernel,
        mesh=mesh,
        compiler_params=pltpu.CompilerParams(use_tc_tiling_on_sc=False),
        out_type=(
            jax.ShapeDtypeStruct((B, D_ITEM), jnp.float32),
            jax.ShapeDtypeStruct((B, D_USER), jnp.float32),
            jax.ShapeDtypeStruct((B * L_SHORT, D_ITEM), jnp.float32),
            jax.ShapeDtypeStruct((B * L_LONG, D_ITEM), jnp.float32),
        ),
        scratch_types=[
            pltpu.VMEM((1, _KL), jnp.int32),
            pltpu.VMEM((1, _KL), jnp.int32),
            pltpu.VMEM((1, _KL), jnp.int32),
            pltpu.VMEM((1, _KI), jnp.int32),
            pltpu.VMEM((1, _KI), jnp.int32),
            pltpu.VMEM((_CHUNK, 64), jnp.float32),
            pltpu.VMEM((_CHUNK, 64), jnp.float32),
            pltpu.VMEM((_CHUNK, 64), jnp.float32),
            pltpu.VMEM((_CHUNK, 64), jnp.float32),
            pltpu.VMEM((_CHUNK, 64), jnp.float32),
            pltpu.SemaphoreType.DMA,
        ],
    )
    def gath(table_h, ig_h, is_h, ic_h, u0_h, u1_h, u2_h, u3_h, u4_h,
             sg_h, ss_h, sc_h, lg_h, ls_h, lc_h,
             xi_h, xu_h, xs_h, xl_h,
             ixa, ixb, ixc, ixd, ixe, ra, rb, rc, rd, re, gsem):
        rbufs = (ra, rb, rc, rd, re)
        wid = lax.axis_index("s") * _NC + lax.axis_index("c")

        def load_idx(srcs, bufs, k):
            for src, buf in zip(srcs, bufs):
                pltpu.sync_copy(src.at[wid], buf.at[:, pl.ds(0, k)])

        def chunk(bufs, out_h, base, coff, csize):
            hs = []
            for t, buf in enumerate(bufs):
                hs.append(pltpu.async_copy(
                    table_h.at[buf.at[0, pl.ds(coff, csize)]],
                    rbufs[t].at[pl.ds(0, csize)],
                    gsem))
            for h in hs:
                h.wait()
            for t in range(len(bufs)):
                pltpu.sync_copy(
                    rbufs[t].at[pl.ds(0, csize)],
                    out_h.at[pl.ds(pl.multiple_of(base + coff, 8), csize),
                             pl.ds(t * 64, 64)])

        def section(srcs, bufs, out_h, k):
            load_idx(srcs, bufs, k)
            base = wid * k
            nch = k // _CHUNK

            def body(j, carry):
                chunk(bufs, out_h, base,
                      pl.multiple_of(j * _CHUNK, _CHUNK), _CHUNK)
                return carry

            if nch > 0:
                lax.fori_loop(0, nch, body, 0)
            rem = k - nch * _CHUNK
            if rem > 0:
                chunk(bufs, out_h, base, nch * _CHUNK, rem)

        section((ig_h, is_h, ic_h), (ixa, ixb, ixc), xi_h, _KI)
        section((u0_h, u1_h, u2_h, u3_h, u4_h),
                (ixa, ixb, ixc, ixd, ixe), xu_h, _KI)
        section((sg_h, ss_h, sc_h), (ixa, ixb, ixc), xs_h, _KS)
        section((lg_h, ls_h, lc_h), (ixa, ixb, ixc), xl_h, _KL)

    return gath(table, ig, ish, ica, u0, u1, u2, u3, u4,
                sg, ss, sc, lg, ls, lc)


def _tc_body(xi_ref, xu_ref, xs_ref, xl_ref, stg_ref, ltg_ref, h_ref,
             sqw_ref, sqb_ref, skw_ref, skb_ref, svw_ref, svb_ref,
             sow_ref, sob_ref,
             lqw_ref, lqb_ref, lkw_ref, lkb_ref, lvw_ref, lvb_ref,
             low_ref, lob_ref,
             w1_ref, b1_ref, g1_ref, be1_ref,
             w2_ref, b2_ref, g2_ref, be2_ref,
             w3_ref, b3_ref, out_ref):
    f32 = jnp.float32
    bB = _BB
    Xi = xi_ref[...]
    Xu = xu_ref[...]
    Xs = xs_ref[...]
    Xl = xl_ref[...]

    ki = lax.broadcasted_iota(jnp.int32, (DQK, NH), 0)
    hi = lax.broadcasted_iota(jnp.int32, (DQK, NH), 1)
    S = (ki // KD == hi).astype(f32)
    ki2 = lax.broadcasted_iota(jnp.int32, (NH, DQK), 1)
    hi2 = lax.broadcasted_iota(jnp.int32, (NH, DQK), 0)
    ST = (ki2 // KD == hi2).astype(f32)

    def mm(a, b):
        return jnp.dot(a, b, preferred_element_type=f32)

    def mha(X, L, pen3, qw, qb, kw, kb, vw, vb, ow, ob):
        qh = mm(Xi, qw) + qb
        kh = mm(X, kw) + kb
        vh = mm(X, vw) + vb
        qr = jnp.broadcast_to(qh.reshape(bB, 1, DQK), (bB, L, DQK))
        qr = qr.reshape(bB * L, DQK)
        sc = mm(kh * qr, S) * (1.0 / math.sqrt(float(KD)))
        sc3 = sc.reshape(bB, L, NH) + pen3
        m = jnp.max(sc3, axis=1, keepdims=True)
        e = jnp.exp(sc3 - m)
        a = e / jnp.sum(e, axis=1, keepdims=True)
        ar = mm(a.reshape(bB * L, NH), ST)
        o = jnp.sum((ar * vh).reshape(bB, L, DQK), axis=1)
        return mm(o, ow) + ob

    svalid = (stg_ref[...] != 0)
    stlen = jnp.sum(svalid.astype(jnp.int32), axis=1, keepdims=True)
    t3 = lax.broadcasted_iota(jnp.int32, (bB, L_SHORT, 1), 1)
    pen_s = jnp.where(t3 < stlen, 0.0, -1e9).astype(f32)
    st_int = mha(Xs, L_SHORT, pen_s,
                 sqw_ref[...], sqb_ref[...], skw_ref[...], skb_ref[...],
                 svw_ref[...], svb_ref[...], sow_ref[...], sob_ref[...])

    H = h_ref[...]
    ic = jnp.sign(mm(Xi, H))
    lc = jnp.sign(mm(Xl, H))
    icr = jnp.broadcast_to(ic.reshape(bB, 1, LSH_DIM), (bB, L_LONG, LSH_DIM))
    eq = (lc.reshape(bB, L_LONG, LSH_DIM) == icr).astype(f32)
    s3 = jnp.sum(eq, axis=2, keepdims=True)
    lvalid = (ltg_ref[...] != 0)
    s3 = jnp.where(lvalid, s3, -1.0)
    l3 = lax.broadcasted_iota(jnp.int32, (bB, L_LONG, 1), 1)
    sel = jnp.zeros((bB, L_LONG, 1), jnp.bool_)
    work = s3
    for _ in range(TOPK):
        mx = jnp.max(work, axis=1, keepdims=True)
        cand = jnp.where(work == mx, l3, L_LONG)
        imin = jnp.min(cand, axis=1, keepdims=True)
        oh = (l3 == imin)
        sel = jnp.logical_or(sel, oh)
        work = jnp.where(oh, -10.0, work)
    pen_l = jnp.where(sel, jnp.where(lvalid, 0.0, -1e9), -2e9).astype(f32)
    lt_int = mha(Xl, L_LONG, pen_l,
                 lqw_ref[...], lqb_ref[...], lkw_ref[...], lkb_ref[...],
                 lvw_ref[...], lvb_ref[...], low_ref[...], lob_ref[...])

    def layernorm(x, g, b):
        mu = jnp.mean(x, axis=1, keepdims=True)
        var = jnp.mean((x - mu) ** 2, axis=1, keepdims=True)
        return (x - mu) / jnp.sqrt(var + 1e-3) * g + b

    xcomb = jnp.concatenate([Xi, Xu, st_int, lt_int], axis=1)
    h1 = jnp.maximum(layernorm(mm(xcomb, w1_ref[...]) + b1_ref[...],
                               g1_ref[...], be1_ref[...]), 0.0)
    h2 = jnp.maximum(layernorm(mm(h1, w2_ref[...]) + b2_ref[...],
                               g2_ref[...], be2_ref[...]), 0.0)
    out_ref[...] = jax.nn.sigmoid(mm(h2, w3_ref[...]) + b3_ref[...])


def _tc_forward(Xi, Xu, Xs, Xl, stg3, ltg3, H,
                sqw, sqb, skw, skb, svw, svb, sow, sob,
                lqw, lqb, lkw, lkb, lvw, lvb, low, lob,
                W1, b1, g1, be1, W2, b2, g2, be2, W3, b3):
    bB = _BB
    grid = (B // bB,)

    def row(i):
        return (i, 0)

    def row3(i):
        return (i, 0, 0)

    def full(i):
        return (0, 0)

    in_specs = [
        pl.BlockSpec((bB, D_ITEM), row),
        pl.BlockSpec((bB, D_USER), row),
        pl.BlockSpec((bB * L_SHORT, D_ITEM), row),
        pl.BlockSpec((bB * L_LONG, D_ITEM), row),
        pl.BlockSpec((bB, L_SHORT, 1), row3),
        pl.BlockSpec((bB, L_LONG, 1), row3),
        pl.BlockSpec((D_ITEM, LSH_DIM), full),
    ]
    for _ in range(2):
        in_specs += [
            pl.BlockSpec((D_ITEM, DQK), full), pl.BlockSpec((1, DQK), full),
            pl.BlockSpec((D_ITEM, DQK), full), pl.BlockSpec((1, DQK), full),
            pl.BlockSpec((D_ITEM, DQK), full), pl.BlockSpec((1, DQK), full),
            pl.BlockSpec((DQK, D_ITEM), full), pl.BlockSpec((1, D_ITEM), full),
        ]
    in_specs += [
        pl.BlockSpec((896, 200), full), pl.BlockSpec((1, 200), full),
        pl.BlockSpec((1, 200), full), pl.BlockSpec((1, 200), full),
        pl.BlockSpec((200, 80), full), pl.BlockSpec((1, 80), full),
        pl.BlockSpec((1, 80), full), pl.BlockSpec((1, 80), full),
        pl.BlockSpec((80, 1), full), pl.BlockSpec((1, 1), full),
    ]
    return pl.pallas_call(
        _tc_body,
        grid=grid,
        in_specs=in_specs,
        out_specs=pl.BlockSpec((bB, 1), row),
        out_shape=jax.ShapeDtypeStruct((B, 1), jnp.float32),
        compiler_params=pltpu.CompilerParams(
            dimension_semantics=("arbitrary",)),
    )(Xi, Xu, Xs, Xl, stg3, ltg3, H,
      sqw, sqb, skw, skb, svw, svb, sow, sob,
      lqw, lqb, lkw, lkb, lvw, lvb, low, lob,
      W1, b1, g1, be1, W2, b2, g2, be2, W3, b3)


def kernel(uid, utag1, utag2, utag3, utag4,
           label_goods_ids, label_shop_ids, label_cate_ids,
           longterm_goods_ids, longterm_shop_ids, longterm_cate_ids,
           shortterm_goods_ids, shortterm_shop_ids, shortterm_cate_ids,
           embed_table, H,
           st_wq, st_bq, st_wk, st_bk, st_wv, st_bv, st_wo, st_bo,
           lt_wq, lt_bq, lt_wk, lt_bk, lt_wv, lt_bv, lt_wo, lt_bo,
           W1, b1, g1, be1, W2, b2, g2, be2, W3, b3):
    i32 = jnp.int32
    lg = label_goods_ids.astype(i32)
    ls = label_shop_ids.astype(i32)
    lc = label_cate_ids.astype(i32)
    stg = shortterm_goods_ids.astype(i32)
    sts = shortterm_shop_ids.astype(i32)
    stc = shortterm_cate_ids.astype(i32)
    ltg = longterm_goods_ids.astype(i32)
    lts = longterm_shop_ids.astype(i32)
    ltc = longterm_cate_ids.astype(i32)

    wi = lambda a, k: a.reshape(_NW, 1, k)
    Xi, Xu, Xs, Xl = _sc_gather_all(
        embed_table,
        wi(lg, _KI), wi(ls, _KI), wi(lc, _KI),
        wi(uid.astype(i32), _KI), wi(utag1.astype(i32), _KI),
        wi(utag2.astype(i32), _KI), wi(utag3.astype(i32), _KI),
        wi(utag4.astype(i32), _KI),
        wi(stg, _KS), wi(sts, _KS), wi(stc, _KS),
        wi(ltg, _KL), wi(lts, _KL), wi(ltc, _KL))

    r2 = lambda v, d: v.reshape(1, d)
    out = _tc_forward(
        Xi, Xu, Xs, Xl,
        stg.reshape(B, L_SHORT, 1), ltg.reshape(B, L_LONG, 1), H,
        st_wq.reshape(D_ITEM, DQK), r2(st_bq, DQK),
        st_wk.reshape(D_ITEM, DQK), r2(st_bk, DQK),
        st_wv.reshape(D_ITEM, DQK), r2(st_bv, DQK),
        st_wo.reshape(DQK, D_ITEM), r2(st_bo, D_ITEM),
        lt_wq.reshape(D_ITEM, DQK), r2(lt_bq, DQK),
        lt_wk.reshape(D_ITEM, DQK), r2(lt_bk, DQK),
        lt_wv.reshape(D_ITEM, DQK), r2(lt_bv, DQK),
        lt_wo.reshape(DQK, D_ITEM), r2(lt_bo, D_ITEM),
        W1, r2(b1, 200), r2(g1, 200), r2(be1, 200),
        W2, r2(b2, 80), r2(g2, 80), r2(be2, 80),
        W3, r2(b3, 1))
    return out


# speedup vs baseline: 15.8384x; 1.7155x over previous
import functools
import math

import jax
import jax.numpy as jnp
from jax import lax
from jax.experimental import pallas as pl
from jax.experimental.pallas import tpu as pltpu
from jax.experimental.pallas import tpu_sc as plsc

B = 1024
E = 64
L_LONG = 200
L_SHORT = 50
LSH_DIM = 16
TOPK = 16
NH = 8
KD = 32
DQK = NH * KD
D_ITEM = 3 * E
D_USER = 5 * E

_NC = 2
_NS = 16
_NW = _NC * _NS

_CHUNK = 128
_NFIRE = 8
_SUPER = _CHUNK * _NFIRE

_BB = 32


_KI = B // _NW
_KS = B * L_SHORT // _NW
_KL = B * L_LONG // _NW


def _sc_gather_all(table, ig, ish, ica, u0, u1, u2, u3, u4,
                   sg, ss, sc, lg, ls, lc):
    mesh = plsc.VectorSubcoreMesh(core_axis_name="c", subcore_axis_name="s")

    @functools.partial(
        pl.kernel,
        mesh=mesh,
        compiler_params=pltpu.CompilerParams(use_tc_tiling_on_sc=False),
        out_type=(
            jax.ShapeDtypeStruct((B, D_ITEM), jnp.float32),
            jax.ShapeDtypeStruct((B, D_USER), jnp.float32),
            jax.ShapeDtypeStruct((B * L_SHORT, D_ITEM), jnp.float32),
            jax.ShapeDtypeStruct((B * L_LONG, D_ITEM), jnp.float32),
        ),
        scratch_types=[
            pltpu.VMEM((1, _KL), jnp.int32),
            pltpu.VMEM((1, _KL), jnp.int32),
            pltpu.VMEM((1, _KL), jnp.int32),
            pltpu.VMEM((1, _KI), jnp.int32),
            pltpu.VMEM((1, _KI), jnp.int32),
            pltpu.VMEM((_CHUNK, 64), jnp.float32),
            pltpu.VMEM((_CHUNK, 64), jnp.float32),
            pltpu.VMEM((_CHUNK, 64), jnp.float32),
            pltpu.VMEM((_CHUNK, 64), jnp.float32),
            pltpu.VMEM((_CHUNK, 64), jnp.float32),
            pltpu.SemaphoreType.DMA,
        ],
    )
    def gath(table_h, ig_h, is_h, ic_h, u0_h, u1_h, u2_h, u3_h, u4_h,
             sg_h, ss_h, sc_h, lg_h, ls_h, lc_h,
             xi_h, xu_h, xs_h, xl_h,
             ixa, ixb, ixc, ixd, ixe, ra, rb, rc, rd, re, gsem):
        rbufs = (ra, rb, rc, rd, re)
        wid = lax.axis_index("s") * _NC + lax.axis_index("c")

        def load_idx(srcs, bufs, k):
            for src, buf in zip(srcs, bufs):
                pltpu.sync_copy(src.at[wid], buf.at[:, pl.ds(0, k)])

        def chunk(bufs, out_h, base, coff, csize):
            hs = []
            for t, buf in enumerate(bufs):
                hs.append(pltpu.async_copy(
                    table_h.at[buf.at[0, pl.ds(coff, csize)]],
                    rbufs[t].at[pl.ds(0, csize)],
                    gsem))
            for h in hs:
                h.wait()
            for t in range(len(bufs)):
                pltpu.sync_copy(
                    rbufs[t].at[pl.ds(0, csize)],
                    out_h.at[pl.ds(pl.multiple_of(base + coff, 8), csize),
                             pl.ds(t * 64, 64)])

        def section(srcs, bufs, out_h, k):
            load_idx(srcs, bufs, k)
            base = wid * k
            nch = k // _CHUNK

            def body(j, carry):
                chunk(bufs, out_h, base,
                      pl.multiple_of(j * _CHUNK, _CHUNK), _CHUNK)
                return carry

            if nch > 0:
                lax.fori_loop(0, nch, body, 0)
            rem = k - nch * _CHUNK
            if rem > 0:
                chunk(bufs, out_h, base, nch * _CHUNK, rem)

        section((ig_h, is_h, ic_h), (ixa, ixb, ixc), xi_h, _KI)
        section((u0_h, u1_h, u2_h, u3_h, u4_h),
                (ixa, ixb, ixc, ixd, ixe), xu_h, _KI)
        section((sg_h, ss_h, sc_h), (ixa, ixb, ixc), xs_h, _KS)
        section((lg_h, ls_h, lc_h), (ixa, ixb, ixc), xl_h, _KL)

    return gath(table, ig, ish, ica, u0, u1, u2, u3, u4,
                sg, ss, sc, lg, ls, lc)


def _tc_body(xi_ref, xu_ref, xs_ref, xl_ref, stg_ref, ltg2_ref, h_ref,
             sqw_ref, sqb_ref, skw_ref, skb_ref, svw_ref, svb_ref,
             sow_ref, sob_ref,
             lqw_ref, lqb_ref, lkw_ref, lkb_ref, lvw_ref, lvb_ref,
             low_ref, lob_ref,
             w1_ref, b1_ref, g1_ref, be1_ref,
             w2_ref, b2_ref, g2_ref, be2_ref,
             w3_ref, b3_ref, out_ref):
    f32 = jnp.float32
    bB = _BB
    Xi = xi_ref[...]
    Xu = xu_ref[...]
    Xs = xs_ref[...]
    Xl = xl_ref[...]

    ki = lax.broadcasted_iota(jnp.int32, (DQK, NH), 0)
    hi = lax.broadcasted_iota(jnp.int32, (DQK, NH), 1)
    S = (ki // KD == hi).astype(f32)
    ki2 = lax.broadcasted_iota(jnp.int32, (NH, DQK), 1)
    hi2 = lax.broadcasted_iota(jnp.int32, (NH, DQK), 0)
    ST = (ki2 // KD == hi2).astype(f32)

    def mm(a, b):
        return jnp.dot(a, b, preferred_element_type=f32)

    def mha(X, L, pen3, qw, qb, kw, kb, vw, vb, ow, ob):
        qh = mm(Xi, qw) + qb
        kh = mm(X, kw) + kb
        vh = mm(X, vw) + vb
        qr = jnp.broadcast_to(qh.reshape(bB, 1, DQK), (bB, L, DQK))
        qr = qr.reshape(bB * L, DQK)
        sc = mm(kh * qr, S) * (1.0 / math.sqrt(float(KD)))
        sc3 = sc.reshape(bB, L, NH) + pen3
        m = jnp.max(sc3, axis=1, keepdims=True)
        e = jnp.exp(sc3 - m)
        a = e / jnp.sum(e, axis=1, keepdims=True)
        ar = mm(a.reshape(bB * L, NH), ST)
        o = jnp.sum((ar * vh).reshape(bB, L, DQK), axis=1)
        return mm(o, ow) + ob

    svalid = (stg_ref[...] != 0)
    stlen = jnp.sum(svalid.astype(jnp.int32), axis=1, keepdims=True)
    t3 = lax.broadcasted_iota(jnp.int32, (bB, L_SHORT, 1), 1)
    pen_s = jnp.where(t3 < stlen, 0.0, -1e9).astype(f32)
    st_int = mha(Xs, L_SHORT, pen_s,
                 sqw_ref[...], sqb_ref[...], skw_ref[...], skb_ref[...],
                 svw_ref[...], svb_ref[...], sow_ref[...], sob_ref[...])

    H = h_ref[...]
    ic = jnp.sign(mm(Xi, H))
    lc = jnp.sign(mm(Xl, H))
    icr = jnp.broadcast_to(ic.reshape(bB, 1, LSH_DIM), (bB, L_LONG, LSH_DIM))
    eq = (lc.reshape(bB, L_LONG, LSH_DIM) == icr).astype(f32)
    s2 = jnp.sum(eq, axis=2)
    lvalid2 = (ltg2_ref[...] != 0)
    s2 = jnp.where(lvalid2, s2, -1.0)
    l2 = lax.broadcasted_iota(jnp.int32, (bB, L_LONG), 1)
    sel = jnp.zeros((bB, L_LONG), jnp.bool_)
    work = s2
    for _ in range(TOPK):
        mx = jnp.max(work, axis=1, keepdims=True)
        cand = jnp.where(work == mx, l2, L_LONG)
        imin = jnp.min(cand, axis=1, keepdims=True)
        oh = (l2 == imin)
        sel = jnp.logical_or(sel, oh)
        work = jnp.where(oh, -10.0, work)
    pen_l2 = jnp.where(sel, jnp.where(lvalid2, 0.0, -1e9), -2e9).astype(f32)
    pen_l = jnp.expand_dims(pen_l2, 2)
    lt_int = mha(Xl, L_LONG, pen_l,
                 lqw_ref[...], lqb_ref[...], lkw_ref[...], lkb_ref[...],
                 lvw_ref[...], lvb_ref[...], low_ref[...], lob_ref[...])

    def layernorm(x, g, b):
        mu = jnp.mean(x, axis=1, keepdims=True)
        var = jnp.mean((x - mu) ** 2, axis=1, keepdims=True)
        return (x - mu) / jnp.sqrt(var + 1e-3) * g + b

    xcomb = jnp.concatenate([Xi, Xu, st_int, lt_int], axis=1)
    h1 = jnp.maximum(layernorm(mm(xcomb, w1_ref[...]) + b1_ref[...],
                               g1_ref[...], be1_ref[...]), 0.0)
    h2 = jnp.maximum(layernorm(mm(h1, w2_ref[...]) + b2_ref[...],
                               g2_ref[...], be2_ref[...]), 0.0)
    out_ref[...] = jax.nn.sigmoid(mm(h2, w3_ref[...]) + b3_ref[...])


def _tc_forward(Xi, Xu, Xs, Xl, stg3, ltg3, H,
                sqw, sqb, skw, skb, svw, svb, sow, sob,
                lqw, lqb, lkw, lkb, lvw, lvb, low, lob,
                W1, b1, g1, be1, W2, b2, g2, be2, W3, b3):
    bB = _BB
    grid = (B // bB,)

    def row(i):
        return (i, 0)

    def row3(i):
        return (i, 0, 0)

    def full(i):
        return (0, 0)

    in_specs = [
        pl.BlockSpec((bB, D_ITEM), row),
        pl.BlockSpec((bB, D_USER), row),
        pl.BlockSpec((bB * L_SHORT, D_ITEM), row),
        pl.BlockSpec((bB * L_LONG, D_ITEM), row),
        pl.BlockSpec((bB, L_SHORT, 1), row3),
        pl.BlockSpec((bB, L_LONG), row),
        pl.BlockSpec((D_ITEM, LSH_DIM), full),
    ]
    for _ in range(2):
        in_specs += [
            pl.BlockSpec((D_ITEM, DQK), full), pl.BlockSpec((1, DQK), full),
            pl.BlockSpec((D_ITEM, DQK), full), pl.BlockSpec((1, DQK), full),
            pl.BlockSpec((D_ITEM, DQK), full), pl.BlockSpec((1, DQK), full),
            pl.BlockSpec((DQK, D_ITEM), full), pl.BlockSpec((1, D_ITEM), full),
        ]
    in_specs += [
        pl.BlockSpec((896, 200), full), pl.BlockSpec((1, 200), full),
        pl.BlockSpec((1, 200), full), pl.BlockSpec((1, 200), full),
        pl.BlockSpec((200, 80), full), pl.BlockSpec((1, 80), full),
        pl.BlockSpec((1, 80), full), pl.BlockSpec((1, 80), full),
        pl.BlockSpec((80, 1), full), pl.BlockSpec((1, 1), full),
    ]
    return pl.pallas_call(
        _tc_body,
        grid=grid,
        in_specs=in_specs,
        out_specs=pl.BlockSpec((bB, 1), row),
        out_shape=jax.ShapeDtypeStruct((B, 1), jnp.float32),
        compiler_params=pltpu.CompilerParams(
            dimension_semantics=("arbitrary",)),
    )(Xi, Xu, Xs, Xl, stg3, ltg3, H,
      sqw, sqb, skw, skb, svw, svb, sow, sob,
      lqw, lqb, lkw, lkb, lvw, lvb, low, lob,
      W1, b1, g1, be1, W2, b2, g2, be2, W3, b3)


def kernel(uid, utag1, utag2, utag3, utag4,
           label_goods_ids, label_shop_ids, label_cate_ids,
           longterm_goods_ids, longterm_shop_ids, longterm_cate_ids,
           shortterm_goods_ids, shortterm_shop_ids, shortterm_cate_ids,
           embed_table, H,
           st_wq, st_bq, st_wk, st_bk, st_wv, st_bv, st_wo, st_bo,
           lt_wq, lt_bq, lt_wk, lt_bk, lt_wv, lt_bv, lt_wo, lt_bo,
           W1, b1, g1, be1, W2, b2, g2, be2, W3, b3):
    i32 = jnp.int32
    lg = label_goods_ids.astype(i32)
    ls = label_shop_ids.astype(i32)
    lc = label_cate_ids.astype(i32)
    stg = shortterm_goods_ids.astype(i32)
    sts = shortterm_shop_ids.astype(i32)
    stc = shortterm_cate_ids.astype(i32)
    ltg = longterm_goods_ids.astype(i32)
    lts = longterm_shop_ids.astype(i32)
    ltc = longterm_cate_ids.astype(i32)

    wi = lambda a, k: a.reshape(_NW, 1, k)
    Xi, Xu, Xs, Xl = _sc_gather_all(
        embed_table,
        wi(lg, _KI), wi(ls, _KI), wi(lc, _KI),
        wi(uid.astype(i32), _KI), wi(utag1.astype(i32), _KI),
        wi(utag2.astype(i32), _KI), wi(utag3.astype(i32), _KI),
        wi(utag4.astype(i32), _KI),
        wi(stg, _KS), wi(sts, _KS), wi(stc, _KS),
        wi(ltg, _KL), wi(lts, _KL), wi(ltc, _KL))

    r2 = lambda v, d: v.reshape(1, d)
    out = _tc_forward(
        Xi, Xu, Xs, Xl,
        stg.reshape(B, L_SHORT, 1), ltg, H,
        st_wq.reshape(D_ITEM, DQK), r2(st_bq, DQK),
        st_wk.reshape(D_ITEM, DQK), r2(st_bk, DQK),
        st_wv.reshape(D_ITEM, DQK), r2(st_bv, DQK),
        st_wo.reshape(DQK, D_ITEM), r2(st_bo, D_ITEM),
        lt_wq.reshape(D_ITEM, DQK), r2(lt_bq, DQK),
        lt_wk.reshape(D_ITEM, DQK), r2(lt_bk, DQK),
        lt_wv.reshape(D_ITEM, DQK), r2(lt_bv, DQK),
        lt_wo.reshape(DQK, D_ITEM), r2(lt_bo, D_ITEM),
        W1, r2(b1, 200), r2(g1, 200), r2(be1, 200),
        W2, r2(b2, 80), r2(g2, 80), r2(be2, 80),
        W3, r2(b3, 1))
    return out


# speedup vs baseline: 16.3471x; 1.0321x over previous
import functools
import math

import jax
import jax.numpy as jnp
from jax import lax
from jax.experimental import pallas as pl
from jax.experimental.pallas import tpu as pltpu
from jax.experimental.pallas import tpu_sc as plsc

B = 1024
E = 64
L_LONG = 200
L_SHORT = 50
LSH_DIM = 16
TOPK = 16
NH = 8
KD = 32
DQK = NH * KD
D_ITEM = 3 * E
D_USER = 5 * E

_NC = 2
_NS = 16
_NW = _NC * _NS

_CHUNK = 128

_BB = 32

_KI = B // _NW
_KS = B * L_SHORT // _NW
_KL = B * L_LONG // _NW

_SC_MESH = dict(core_axis_name="c", subcore_axis_name="s")


def _make_section_runner(table_h, rbufs, gsem, wid):
    def chunk(bufs, out_h, base, coff, csize):
        hs = []
        for t, buf in enumerate(bufs):
            hs.append(pltpu.async_copy(
                table_h.at[buf.at[0, pl.ds(coff, csize)]],
                rbufs[t].at[pl.ds(0, csize)],
                gsem))
        for h in hs:
            h.wait()
        for t in range(len(bufs)):
            pltpu.sync_copy(
                rbufs[t].at[pl.ds(0, csize)],
                out_h.at[pl.ds(pl.multiple_of(base + coff, 8), csize),
                         pl.ds(t * 64, 64)])

    def load_idx(srcs, bufs, k):
        for src, buf in zip(srcs, bufs):
            pltpu.sync_copy(src.at[wid], buf.at[:, pl.ds(0, k)])

    def section(srcs, bufs, out_h, k):
        load_idx(srcs, bufs, k)
        base = wid * k
        nch = k // _CHUNK

        def body(j, carry):
            chunk(bufs, out_h, base,
                  pl.multiple_of(j * _CHUNK, _CHUNK), _CHUNK)
            return carry

        if nch > 0:
            lax.fori_loop(0, nch, body, 0)
        rem = k - nch * _CHUNK
        if rem > 0:
            chunk(bufs, out_h, base, nch * _CHUNK, rem)

    return section


def _sc_gather_short(table, ig, ish, ica, u0, u1, u2, u3, u4, sg, ss, sc):
    mesh = plsc.VectorSubcoreMesh(**_SC_MESH)

    @functools.partial(
        pl.kernel,
        mesh=mesh,
        compiler_params=pltpu.CompilerParams(use_tc_tiling_on_sc=False),
        out_type=(
            jax.ShapeDtypeStruct((B, D_ITEM), jnp.float32),
            jax.ShapeDtypeStruct((B, D_USER), jnp.float32),
            jax.ShapeDtypeStruct((B * L_SHORT, D_ITEM), jnp.float32),
        ),
        scratch_types=[
            pltpu.VMEM((1, _KS), jnp.int32),
            pltpu.VMEM((1, _KS), jnp.int32),
            pltpu.VMEM((1, _KS), jnp.int32),
            pltpu.VMEM((1, _KI), jnp.int32),
            pltpu.VMEM((1, _KI), jnp.int32),
            pltpu.VMEM((_CHUNK, 64), jnp.float32),
            pltpu.VMEM((_CHUNK, 64), jnp.float32),
            pltpu.VMEM((_CHUNK, 64), jnp.float32),
            pltpu.VMEM((_CHUNK, 64), jnp.float32),
            pltpu.VMEM((_CHUNK, 64), jnp.float32),
            pltpu.SemaphoreType.DMA,
        ],
    )
    def gath(table_h, ig_h, is_h, ic_h, u0_h, u1_h, u2_h, u3_h, u4_h,
             sg_h, ss_h, sc_h, xi_h, xu_h, xs_h,
             ixa, ixb, ixc, ixd, ixe, ra, rb, rc, rd, re, gsem):
        wid = lax.axis_index("s") * _NC + lax.axis_index("c")
        section = _make_section_runner(table_h, (ra, rb, rc, rd, re),
                                       gsem, wid)
        section((ig_h, is_h, ic_h), (ixa, ixb, ixc), xi_h, _KI)
        section((u0_h, u1_h, u2_h, u3_h, u4_h),
                (ixa, ixb, ixc, ixd, ixe), xu_h, _KI)
        section((sg_h, ss_h, sc_h), (ixa, ixb, ixc), xs_h, _KS)

    return gath(table, ig, ish, ica, u0, u1, u2, u3, u4, sg, ss, sc)


def _sc_gather_long(table, lg, ls, lc):
    mesh = plsc.VectorSubcoreMesh(**_SC_MESH)

    @functools.partial(
        pl.kernel,
        mesh=mesh,
        compiler_params=pltpu.CompilerParams(use_tc_tiling_on_sc=False),
        out_type=jax.ShapeDtypeStruct((B * L_LONG, D_ITEM), jnp.float32),
        scratch_types=[
            pltpu.VMEM((1, _KL), jnp.int32),
            pltpu.VMEM((1, _KL), jnp.int32),
            pltpu.VMEM((1, _KL), jnp.int32),
            pltpu.VMEM((_CHUNK, 64), jnp.float32),
            pltpu.VMEM((_CHUNK, 64), jnp.float32),
            pltpu.VMEM((_CHUNK, 64), jnp.float32),
            pltpu.SemaphoreType.DMA,
        ],
    )
    def gath(table_h, lg_h, ls_h, lc_h, xl_h,
             ixa, ixb, ixc, ra, rb, rc, gsem):
        wid = lax.axis_index("s") * _NC + lax.axis_index("c")
        section = _make_section_runner(table_h, (ra, rb, rc), gsem, wid)
        section((lg_h, ls_h, lc_h), (ixa, ixb, ixc), xl_h, _KL)

    return gath(table, lg, ls, lc)


def _head_mats():
    f32 = jnp.float32
    ki = lax.broadcasted_iota(jnp.int32, (DQK, NH), 0)
    hi = lax.broadcasted_iota(jnp.int32, (DQK, NH), 1)
    S = (ki // KD == hi).astype(f32)
    ki2 = lax.broadcasted_iota(jnp.int32, (NH, DQK), 1)
    hi2 = lax.broadcasted_iota(jnp.int32, (NH, DQK), 0)
    ST = (ki2 // KD == hi2).astype(f32)
    return S, ST


def _mm(a, b):
    return jnp.dot(a, b, preferred_element_type=jnp.float32)


def _mha(Xi, X, L, pen3, qw, qb, kw, kb, vw, vb, ow, ob, S, ST):
    bB = _BB
    qh = _mm(Xi, qw) + qb
    kh = _mm(X, kw) + kb
    vh = _mm(X, vw) + vb
    qr = jnp.broadcast_to(qh.reshape(bB, 1, DQK), (bB, L, DQK))
    qr = qr.reshape(bB * L, DQK)
    sc = _mm(kh * qr, S) * (1.0 / math.sqrt(float(KD)))
    sc3 = sc.reshape(bB, L, NH) + pen3
    m = jnp.max(sc3, axis=1, keepdims=True)
    e = jnp.exp(sc3 - m)
    a = e / jnp.sum(e, axis=1, keepdims=True)
    ar = _mm(a.reshape(bB * L, NH), ST)
    o = jnp.sum((ar * vh).reshape(bB, L, DQK), axis=1)
    return _mm(o, ow) + ob


def _tc1_body(xi_ref, xs_ref, stg_ref,
              sqw_ref, sqb_ref, skw_ref, skb_ref, svw_ref, svb_ref,
              sow_ref, sob_ref, out_ref):
    f32 = jnp.float32
    bB = _BB
    S, ST = _head_mats()
    Xi = xi_ref[...]
    Xs = xs_ref[...]
    svalid = (stg_ref[...] != 0)
    stlen = jnp.sum(svalid.astype(jnp.int32), axis=1, keepdims=True)
    t2 = lax.broadcasted_iota(jnp.int32, (bB, L_SHORT), 1)
    pen_s2 = jnp.where(t2 < stlen, 0.0, -1e9).astype(f32)
    pen_s = jnp.expand_dims(pen_s2, 2)
    out_ref[...] = _mha(
        Xi, Xs, L_SHORT, pen_s,
        sqw_ref[...], sqb_ref[...], skw_ref[...], skb_ref[...],
        svw_ref[...], svb_ref[...], sow_ref[...], sob_ref[...], S, ST)


def _tc2_body(xi_ref, xu_ref, xl_ref, ltg2_ref, stint_ref, h_ref,
              lqw_ref, lqb_ref, lkw_ref, lkb_ref, lvw_ref, lvb_ref,
              low_ref, lob_ref,
              w1_ref, b1_ref, g1_ref, be1_ref,
              w2_ref, b2_ref, g2_ref, be2_ref,
              w3_ref, b3_ref, out_ref):
    f32 = jnp.float32
    bB = _BB
    S, ST = _head_mats()
    Xi = xi_ref[...]
    Xu = xu_ref[...]
    Xl = xl_ref[...]

    H = h_ref[...]
    ic = jnp.sign(_mm(Xi, H))
    lc = jnp.sign(_mm(Xl, H))
    icr = jnp.broadcast_to(ic.reshape(bB, 1, LSH_DIM), (bB, L_LONG, LSH_DIM))
    eq = (lc.reshape(bB, L_LONG, LSH_DIM) == icr).astype(f32)
    s2 = jnp.sum(eq, axis=2)
    lvalid2 = (ltg2_ref[...] != 0)
    s2 = jnp.where(lvalid2, s2, -1.0)
    l2 = lax.broadcasted_iota(jnp.int32, (bB, L_LONG), 1)
    sel = jnp.zeros((bB, L_LONG), jnp.bool_)
    work = s2
    for _ in range(TOPK):
        mx = jnp.max(work, axis=1, keepdims=True)
        cand = jnp.where(work == mx, l2, L_LONG)
        imin = jnp.min(cand, axis=1, keepdims=True)
        oh = (l2 == imin)
        sel = jnp.logical_or(sel, oh)
        work = jnp.where(oh, -10.0, work)
    pen_l2 = jnp.where(sel, jnp.where(lvalid2, 0.0, -1e9), -2e9).astype(f32)
    pen_l = jnp.expand_dims(pen_l2, 2)

    lt_int = _mha(Xi, Xl, L_LONG, pen_l,
                  lqw_ref[...], lqb_ref[...], lkw_ref[...], lkb_ref[...],
                  lvw_ref[...], lvb_ref[...], low_ref[...], lob_ref[...],
                  S, ST)

    def layernorm(x, g, b):
        mu = jnp.mean(x, axis=1, keepdims=True)
        var = jnp.mean((x - mu) ** 2, axis=1, keepdims=True)
        return (x - mu) / jnp.sqrt(var + 1e-3) * g + b

    xcomb = jnp.concatenate([Xi, Xu, stint_ref[...], lt_int], axis=1)
    h1 = jnp.maximum(layernorm(_mm(xcomb, w1_ref[...]) + b1_ref[...],
                               g1_ref[...], be1_ref[...]), 0.0)
    h2 = jnp.maximum(layernorm(_mm(h1, w2_ref[...]) + b2_ref[...],
                               g2_ref[...], be2_ref[...]), 0.0)
    out_ref[...] = jax.nn.sigmoid(_mm(h2, w3_ref[...]) + b3_ref[...])


def _row(i):
    return (i, 0)


def _full(i):
    return (0, 0)


def _tc1(Xi, Xs, stg, sqw, sqb, skw, skb, svw, svb, sow, sob):
    bB = _BB
    in_specs = [
        pl.BlockSpec((bB, D_ITEM), _row),
        pl.BlockSpec((bB * L_SHORT, D_ITEM), _row),
        pl.BlockSpec((bB, L_SHORT), _row),
        pl.BlockSpec((D_ITEM, DQK), _full), pl.BlockSpec((1, DQK), _full),
        pl.BlockSpec((D_ITEM, DQK), _full), pl.BlockSpec((1, DQK), _full),
        pl.BlockSpec((D_ITEM, DQK), _full), pl.BlockSpec((1, DQK), _full),
        pl.BlockSpec((DQK, D_ITEM), _full), pl.BlockSpec((1, D_ITEM), _full),
    ]
    return pl.pallas_call(
        _tc1_body,
        grid=(B // bB,),
        in_specs=in_specs,
        out_specs=pl.BlockSpec((bB, D_ITEM), _row),
        out_shape=jax.ShapeDtypeStruct((B, D_ITEM), jnp.float32),
        compiler_params=pltpu.CompilerParams(
            dimension_semantics=("arbitrary",)),
    )(Xi, Xs, stg, sqw, sqb, skw, skb, svw, svb, sow, sob)


def _tc2(Xi, Xu, Xl, ltg, st_int, H,
         lqw, lqb, lkw, lkb, lvw, lvb, low, lob,
         W1, b1, g1, be1, W2, b2, g2, be2, W3, b3):
    bB = _BB
    in_specs = [
        pl.BlockSpec((bB, D_ITEM), _row),
        pl.BlockSpec((bB, D_USER), _row),
        pl.BlockSpec((bB * L_LONG, D_ITEM), _row),
        pl.BlockSpec((bB, L_LONG), _row),
        pl.BlockSpec((bB, D_ITEM), _row),
        pl.BlockSpec((D_ITEM, LSH_DIM), _full),
        pl.BlockSpec((D_ITEM, DQK), _full), pl.BlockSpec((1, DQK), _full),
        pl.BlockSpec((D_ITEM, DQK), _full), pl.BlockSpec((1, DQK), _full),
        pl.BlockSpec((D_ITEM, DQK), _full), pl.BlockSpec((1, DQK), _full),
        pl.BlockSpec((DQK, D_ITEM), _full), pl.BlockSpec((1, D_ITEM), _full),
        pl.BlockSpec((896, 200), _full), pl.BlockSpec((1, 200), _full),
        pl.BlockSpec((1, 200), _full), pl.BlockSpec((1, 200), _full),
        pl.BlockSpec((200, 80), _full), pl.BlockSpec((1, 80), _full),
        pl.BlockSpec((1, 80), _full), pl.BlockSpec((1, 80), _full),
        pl.BlockSpec((80, 1), _full), pl.BlockSpec((1, 1), _full),
    ]
    return pl.pallas_call(
        _tc2_body,
        grid=(B // bB,),
        in_specs=in_specs,
        out_specs=pl.BlockSpec((bB, 1), _row),
        out_shape=jax.ShapeDtypeStruct((B, 1), jnp.float32),
        compiler_params=pltpu.CompilerParams(
            dimension_semantics=("arbitrary",)),
    )(Xi, Xu, Xl, ltg, st_int, H,
      lqw, lqb, lkw, lkb, lvw, lvb, low, lob,
      W1, b1, g1, be1, W2, b2, g2, be2, W3, b3)


def kernel(uid, utag1, utag2, utag3, utag4,
           label_goods_ids, label_shop_ids, label_cate_ids,
           longterm_goods_ids, longterm_shop_ids, longterm_cate_ids,
           shortterm_goods_ids, shortterm_shop_ids, shortterm_cate_ids,
           embed_table, H,
           st_wq, st_bq, st_wk, st_bk, st_wv, st_bv, st_wo, st_bo,
           lt_wq, lt_bq, lt_wk, lt_bk, lt_wv, lt_bv, lt_wo, lt_bo,
           W1, b1, g1, be1, W2, b2, g2, be2, W3, b3):
    i32 = jnp.int32
    lg = label_goods_ids.astype(i32)
    ls = label_shop_ids.astype(i32)
    lc = label_cate_ids.astype(i32)
    stg = shortterm_goods_ids.astype(i32)
    sts = shortterm_shop_ids.astype(i32)
    stc = shortterm_cate_ids.astype(i32)
    ltg = longterm_goods_ids.astype(i32)
    lts = longterm_shop_ids.astype(i32)
    ltc = longterm_cate_ids.astype(i32)

    wi = lambda a, k: a.reshape(_NW, 1, k)
    Xl = _sc_gather_long(embed_table,
                         wi(ltg, _KL), wi(lts, _KL), wi(ltc, _KL))
    Xi, Xu, Xs = _sc_gather_short(
        embed_table,
        wi(lg, _KI), wi(ls, _KI), wi(lc, _KI),
        wi(uid.astype(i32), _KI), wi(utag1.astype(i32), _KI),
        wi(utag2.astype(i32), _KI), wi(utag3.astype(i32), _KI),
        wi(utag4.astype(i32), _KI),
        wi(stg, _KS), wi(sts, _KS), wi(stc, _KS))

    r2 = lambda v, d: v.reshape(1, d)
    st_int = _tc1(Xi, Xs, stg,
                  st_wq.reshape(D_ITEM, DQK), r2(st_bq, DQK),
                  st_wk.reshape(D_ITEM, DQK), r2(st_bk, DQK),
                  st_wv.reshape(D_ITEM, DQK), r2(st_bv, DQK),
                  st_wo.reshape(DQK, D_ITEM), r2(st_bo, D_ITEM))
    out = _tc2(Xi, Xu, Xl, ltg, st_int, H,
               lt_wq.reshape(D_ITEM, DQK), r2(lt_bq, DQK),
               lt_wk.reshape(D_ITEM, DQK), r2(lt_bk, DQK),
               lt_wv.reshape(D_ITEM, DQK), r2(lt_bv, DQK),
               lt_wo.reshape(DQK, D_ITEM), r2(lt_bo, D_ITEM),
               W1, r2(b1, 200), r2(g1, 200), r2(be1, 200),
               W2, r2(b2, 80), r2(g2, 80), r2(be2, 80),
               W3, r2(b3, 1))
    return out
